# Initial kernel scaffold; baseline (speedup 1.0000x reference)
import jax
import jax.numpy as jnp
from jax.experimental import pallas as pl


def kernel(prob, u, kk, alias):
    raise NotImplementedError("write your pallas kernel here")


# speedup vs baseline: 130.9504x; 130.9504x over previous
import functools

import jax
import jax.numpy as jnp
from jax import lax
from jax.experimental import pallas as pl
from jax.experimental.pallas import tpu as pltpu
from jax.experimental.pallas import tpu_sc as plsc

N = 4194304
K = 100000
NC = 2
NS = 16
NW = NC * NS
NPW = N // NW
C = 2048
NCHUNK = NPW // C
L = 16


def _sc_body(prob_hbm, alias_hbm, kk_hbm, u_hbm, out_hbm,
             idx_v, u_v, p_v, a_v, out_v, sem):
    wid = lax.axis_index("s") * jnp.int32(NC) + lax.axis_index("c")
    wbase = wid * jnp.int32(NPW)

    def chunk_body(i, carry):
        base = wbase + i * jnp.int32(C)
        pltpu.sync_copy(kk_hbm.at[pl.ds(base, C)], idx_v)
        pltpu.sync_copy(u_hbm.at[pl.ds(base, C)], u_v)
        cp_p = pltpu.async_copy(prob_hbm.at[idx_v], p_v, sem)
        cp_a = pltpu.async_copy(alias_hbm.at[idx_v], a_v, sem)
        cp_p.wait()
        cp_a.wait()

        def vec_body(j, carry2):
            off = j * jnp.int32(L)
            p = p_v[pl.ds(off, L)]
            av = a_v[pl.ds(off, L)]
            kkv = idx_v[pl.ds(off, L)]
            uv = u_v[pl.ds(off, L)]
            out_v[pl.ds(off, L)] = jnp.where(uv < p, kkv, av)
            return carry2

        lax.fori_loop(jnp.int32(0), jnp.int32(C // L), vec_body, 0)
        pltpu.sync_copy(out_v, out_hbm.at[pl.ds(base, C)])
        return carry

    lax.fori_loop(jnp.int32(0), jnp.int32(NCHUNK), chunk_body, 0)


_mesh = plsc.VectorSubcoreMesh(core_axis_name="c", subcore_axis_name="s")

_draw = functools.partial(
    pl.kernel,
    mesh=_mesh,
    out_type=jax.ShapeDtypeStruct((N,), jnp.int32),
    scratch_types=[
        pltpu.VMEM((C,), jnp.int32),
        pltpu.VMEM((C,), jnp.float32),
        pltpu.VMEM((C,), jnp.float32),
        pltpu.VMEM((C,), jnp.int32),
        pltpu.VMEM((C,), jnp.int32),
        pltpu.SemaphoreType.DMA,
    ],
)(_sc_body)


def kernel(prob, u, kk, alias):
    kk32 = kk.astype(jnp.int32)
    alias32 = alias.astype(jnp.int32)
    out32 = _draw(prob, alias32, kk32, u)
    return out32.astype(kk.dtype)


# speedup vs baseline: 149.1361x; 1.1389x over previous
import functools

import jax
import jax.numpy as jnp
from jax import lax
from jax.experimental import pallas as pl
from jax.experimental.pallas import tpu as pltpu
from jax.experimental.pallas import tpu_sc as plsc

N = 4194304
K = 100000
NC = 2
NS = 16
NW = NC * NS
NPW = N // NW
C = 8192
NCHUNK = NPW // C
L = 16
U = 8


def _sc_body(prob_hbm, alias_hbm, kk_hbm, u_hbm, out_hbm,
             idx0, idx1, u0, u1, p0, p1, a0, a1, o0, o1,
             si0, si1, sg0, sg1, so0, so1):
    idx = (idx0, idx1)
    uu = (u0, u1)
    pp = (p0, p1)
    aa = (a0, a1)
    oo = (o0, o1)
    si = (si0, si1)
    sg = (sg0, sg1)
    so = (so0, so1)

    wid = lax.axis_index("s") * jnp.int32(NC) + lax.axis_index("c")
    wbase = wid * jnp.int32(NPW)

    def base_of(g):
        return wbase + jnp.int32(g * C)

    def issue_in(g):
        b = g % 2
        cp_k = pltpu.make_async_copy(kk_hbm.at[pl.ds(base_of(g), C)], idx[b], si[b])
        cp_u = pltpu.make_async_copy(u_hbm.at[pl.ds(base_of(g), C)], uu[b], si[b])
        cp_k.start()
        cp_u.start()
        return cp_k, cp_u

    def issue_gather(g):
        b = g % 2
        cp_p = pltpu.make_async_copy(prob_hbm.at[idx[b]], pp[b], sg[b])
        cp_a = pltpu.make_async_copy(alias_hbm.at[idx[b]], aa[b], sg[b])
        cp_p.start()
        cp_a.start()
        return cp_p, cp_a

    def issue_out(g):
        b = g % 2
        cp_o = pltpu.make_async_copy(oo[b], out_hbm.at[pl.ds(base_of(g), C)], so[b])
        cp_o.start()
        return cp_o

    def compute(g):
        b = g % 2

        def vec_body(j, carry):
            off0 = j * jnp.int32(L * U)
            for k in range(U):
                off = off0 + jnp.int32(k * L)
                p = pp[b][pl.ds(off, L)]
                av = aa[b][pl.ds(off, L)]
                kkv = idx[b][pl.ds(off, L)]
                uv = uu[b][pl.ds(off, L)]
                oo[b][pl.ds(off, L)] = jnp.where(uv < p, kkv, av)
            return carry

        lax.fori_loop(jnp.int32(0), jnp.int32(C // (L * U)), vec_body, 0)

    cp_in = {}
    cp_g = {}
    cp_out = {}

    cp_in[0] = issue_in(0)
    cp_in[1] = issue_in(1)
    cp_in[0][0].wait()
    cp_in[0][1].wait()
    cp_g[0] = issue_gather(0)

    for g in range(NCHUNK):
        if g + 1 < NCHUNK:
            cp_in[g + 1][0].wait()
            cp_in[g + 1][1].wait()
            cp_g[g + 1] = issue_gather(g + 1)
        cp_g[g][0].wait()
        cp_g[g][1].wait()
        if g >= 2:
            cp_out[g - 2].wait()
        compute(g)
        cp_out[g] = issue_out(g)
        if g + 2 < NCHUNK:
            cp_in[g + 2] = issue_in(g + 2)

    cp_out[NCHUNK - 2].wait()
    cp_out[NCHUNK - 1].wait()


_mesh = plsc.VectorSubcoreMesh(core_axis_name="c", subcore_axis_name="s")

_draw = functools.partial(
    pl.kernel,
    mesh=_mesh,
    out_type=jax.ShapeDtypeStruct((N,), jnp.int32),
    scratch_types=[
        pltpu.VMEM((C,), jnp.int32), pltpu.VMEM((C,), jnp.int32),
        pltpu.VMEM((C,), jnp.float32), pltpu.VMEM((C,), jnp.float32),
        pltpu.VMEM((C,), jnp.float32), pltpu.VMEM((C,), jnp.float32),
        pltpu.VMEM((C,), jnp.int32), pltpu.VMEM((C,), jnp.int32),
        pltpu.VMEM((C,), jnp.int32), pltpu.VMEM((C,), jnp.int32),
        pltpu.SemaphoreType.DMA, pltpu.SemaphoreType.DMA,
        pltpu.SemaphoreType.DMA, pltpu.SemaphoreType.DMA,
        pltpu.SemaphoreType.DMA, pltpu.SemaphoreType.DMA,
    ],
)(_sc_body)


def kernel(prob, u, kk, alias):
    kk32 = kk.astype(jnp.int32)
    alias32 = alias.astype(jnp.int32)
    out32 = _draw(prob, alias32, kk32, u)
    return out32.astype(kk.dtype)


# speedup vs baseline: 223.7786x; 1.5005x over previous
import functools

import jax
import jax.numpy as jnp
from jax import lax
from jax.experimental import pallas as pl
from jax.experimental.pallas import tpu as pltpu
from jax.experimental.pallas import tpu_sc as plsc

N = 4194304
K = 100000
NC = 2
NS = 16
NW = NC * NS
NPW = N // NW
C = 8192
NCHUNK = NPW // C
L = 16
U = 8
P = 2000
NP = K // P


def _sc_body(prob_hbm, alias_hbm, kk_hbm, u_hbm, out_hbm,
             idx0, idx1, u0, u1, p0, p1, a0, a1, o0, o1,
             prob_sp, alias_sp,
             si0, si1, sg0, sg1, so0, so1):
    idx = (idx0, idx1)
    uu = (u0, u1)
    pp = (p0, p1)
    aa = (a0, a1)
    oo = (o0, o1)
    si = (si0, si1)
    sg = (sg0, sg1)
    so = (so0, so1)

    sid = lax.axis_index("s")
    wid = sid * jnp.int32(NC) + lax.axis_index("c")
    wbase = wid * jnp.int32(NPW)

    for g in range(NP):
        @pl.when(sid == jnp.int32(g % NS))
        def _():
            pltpu.sync_copy(prob_hbm.at[pl.ds(g * P, P)],
                            uu[0].at[pl.ds(0, P)])
            pltpu.sync_copy(uu[0].at[pl.ds(0, P)],
                            prob_sp.at[pl.ds(g * P, P)])
            pltpu.sync_copy(alias_hbm.at[pl.ds(g * P, P)],
                            idx[0].at[pl.ds(0, P)])
            pltpu.sync_copy(idx[0].at[pl.ds(0, P)],
                            alias_sp.at[pl.ds(g * P, P)])

    plsc.subcore_barrier()

    def base_of(g):
        return wbase + jnp.int32(g * C)

    def issue_in(g):
        b = g % 2
        cp_k = pltpu.make_async_copy(
            kk_hbm.at[pl.ds(base_of(g), C)], idx[b], si[b])
        cp_u = pltpu.make_async_copy(
            u_hbm.at[pl.ds(base_of(g), C)], uu[b], si[b])
        cp_k.start()
        cp_u.start()
        return cp_k, cp_u

    def issue_gather(g):
        b = g % 2
        cp_p = pltpu.make_async_copy(prob_sp.at[idx[b]], pp[b], sg[b])
        cp_a = pltpu.make_async_copy(alias_sp.at[idx[b]], aa[b], sg[b])
        cp_p.start()
        cp_a.start()
        return cp_p, cp_a

    def issue_out(g):
        b = g % 2
        cp_o = pltpu.make_async_copy(
            oo[b], out_hbm.at[pl.ds(base_of(g), C)], so[b])
        cp_o.start()
        return cp_o

    def compute(g):
        b = g % 2

        def vec_body(j, carry):
            off0 = j * jnp.int32(L * U)
            for k in range(U):
                off = off0 + jnp.int32(k * L)
                p = pp[b][pl.ds(off, L)]
                av = aa[b][pl.ds(off, L)]
                kkv = idx[b][pl.ds(off, L)]
                uv = uu[b][pl.ds(off, L)]
                oo[b][pl.ds(off, L)] = jnp.where(uv < p, kkv, av)
            return carry

        lax.fori_loop(jnp.int32(0), jnp.int32(C // (L * U)), vec_body, 0)

    cp_in = {}
    cp_g = {}
    cp_out = {}

    cp_in[0] = issue_in(0)
    cp_in[1] = issue_in(1)
    cp_in[0][0].wait()
    cp_in[0][1].wait()
    cp_g[0] = issue_gather(0)

    for g in range(NCHUNK):
        if g + 1 < NCHUNK:
            cp_in[g + 1][0].wait()
            cp_in[g + 1][1].wait()
            cp_g[g + 1] = issue_gather(g + 1)
        cp_g[g][0].wait()
        cp_g[g][1].wait()
        if g >= 2:
            cp_out[g - 2].wait()
        compute(g)
        cp_out[g] = issue_out(g)
        if g + 2 < NCHUNK:
            cp_in[g + 2] = issue_in(g + 2)

    cp_out[NCHUNK - 2].wait()
    cp_out[NCHUNK - 1].wait()


_mesh = plsc.VectorSubcoreMesh(core_axis_name="c", subcore_axis_name="s")

_draw = functools.partial(
    pl.kernel,
    mesh=_mesh,
    out_type=jax.ShapeDtypeStruct((N,), jnp.int32),
    scratch_types=[
        pltpu.VMEM((C,), jnp.int32), pltpu.VMEM((C,), jnp.int32),
        pltpu.VMEM((C,), jnp.float32), pltpu.VMEM((C,), jnp.float32),
        pltpu.VMEM((C,), jnp.float32), pltpu.VMEM((C,), jnp.float32),
        pltpu.VMEM((C,), jnp.int32), pltpu.VMEM((C,), jnp.int32),
        pltpu.VMEM((C,), jnp.int32), pltpu.VMEM((C,), jnp.int32),
        pltpu.VMEM_SHARED((K,), jnp.float32),
        pltpu.VMEM_SHARED((K,), jnp.int32),
        pltpu.SemaphoreType.DMA, pltpu.SemaphoreType.DMA,
        pltpu.SemaphoreType.DMA, pltpu.SemaphoreType.DMA,
        pltpu.SemaphoreType.DMA, pltpu.SemaphoreType.DMA,
    ],
)(_sc_body)


def kernel(prob, u, kk, alias):
    kk32 = lax.bitcast_convert_type(
        lax.convert_element_type(kk, jnp.uint32), jnp.int32)
    alias32 = lax.bitcast_convert_type(
        lax.convert_element_type(alias, jnp.uint32), jnp.int32)
    out32 = _draw(prob, alias32, kk32, u)
    return lax.convert_element_type(
        lax.bitcast_convert_type(out32, jnp.uint32), jnp.int64)


# speedup vs baseline: 233.5575x; 1.0437x over previous
import functools

import jax
import jax.numpy as jnp
from jax import lax
from jax.experimental import pallas as pl
from jax.experimental.pallas import tpu as pltpu
from jax.experimental.pallas import tpu_sc as plsc

N = 4194304
K = 100000
NC = 2
NS = 16
NW = NC * NS
NPW = N // NW
C = 8192
NCHUNK = NPW // C
L = 16
U = 8
P = 2000
NP = K // P


def _sc_body(prob_hbm, alias_hbm, kk_hbm, u_hbm, out_hbm,
             idx0, idx1, u0, u1, p0, p1, a0, a1, o0, o1,
             prob_sp, alias_sp,
             si0, si1, sg0, sg1, so0, so1):
    idx = (idx0, idx1)
    uu = (u0, u1)
    pp = (p0, p1)
    aa = (a0, a1)
    oo = (o0, o1)
    si = (si0, si1)
    sg = (sg0, sg1)
    so = (so0, so1)

    sid = lax.axis_index("s")
    wid = sid * jnp.int32(NC) + lax.axis_index("c")
    wbase = wid * jnp.int32(NPW)

    for g in range(NP):
        @pl.when(sid == jnp.int32(g % NS))
        def _():
            pltpu.sync_copy(prob_hbm.at[pl.ds(g * P, P)],
                            uu[0].at[pl.ds(0, P)])
            pltpu.sync_copy(uu[0].at[pl.ds(0, P)],
                            prob_sp.at[pl.ds(g * P, P)])
            pltpu.sync_copy(alias_hbm.at[pl.ds(g * P, P)],
                            idx[0].at[pl.ds(0, P)])
            pltpu.sync_copy(idx[0].at[pl.ds(0, P)],
                            alias_sp.at[pl.ds(g * P, P)])

    plsc.subcore_barrier()

    def base_of(g):
        return wbase + jnp.int32(g * C)

    def issue_in(g):
        b = g % 2
        cp_k = pltpu.make_async_copy(
            kk_hbm.at[pl.ds(base_of(g), C)], idx[b], si[b])
        cp_u = pltpu.make_async_copy(
            u_hbm.at[pl.ds(base_of(g), C)], uu[b], si[b])
        cp_k.start()
        cp_u.start()
        return cp_k, cp_u

    def issue_gather(g):
        b = g % 2
        cp_p = pltpu.make_async_copy(prob_sp.at[idx[b]], pp[b], sg[b])
        cp_a = pltpu.make_async_copy(alias_sp.at[idx[b]], aa[b], sg[b])
        cp_p.start()
        cp_a.start()
        return cp_p, cp_a

    def issue_out(g):
        b = g % 2
        cp_o = pltpu.make_async_copy(
            oo[b], out_hbm.at[pl.ds(base_of(g), C)], so[b])
        cp_o.start()
        return cp_o

    def compute(g):
        b = g % 2

        def vec_body(j, carry):
            off0 = j * jnp.int32(L * U)
            for k in range(U):
                off = off0 + jnp.int32(k * L)
                p = pp[b][pl.ds(off, L)]
                av = aa[b][pl.ds(off, L)]
                kkv = idx[b][pl.ds(off, L)]
                uv = uu[b][pl.ds(off, L)]
                oo[b][pl.ds(off, L)] = jnp.where(uv < p, kkv, av)
            return carry

        lax.fori_loop(jnp.int32(0), jnp.int32(C // (L * U)), vec_body, 0)

    cp_in = {}
    cp_g = {}
    cp_out = {}

    cp_in[0] = issue_in(0)
    cp_in[1] = issue_in(1)
    cp_in[0][0].wait()
    cp_in[0][1].wait()
    cp_g[0] = issue_gather(0)

    for g in range(NCHUNK):
        if g + 1 < NCHUNK:
            cp_in[g + 1][0].wait()
            cp_in[g + 1][1].wait()
            cp_g[g + 1] = issue_gather(g + 1)
        cp_g[g][0].wait()
        cp_g[g][1].wait()
        if g >= 2:
            cp_out[g - 2].wait()
        compute(g)
        cp_out[g] = issue_out(g)
        if g + 2 < NCHUNK:
            cp_in[g + 2] = issue_in(g + 2)

    cp_out[NCHUNK - 2].wait()
    cp_out[NCHUNK - 1].wait()


_mesh = plsc.VectorSubcoreMesh(core_axis_name="c", subcore_axis_name="s")

_draw = functools.partial(
    pl.kernel,
    mesh=_mesh,
    out_type=jax.ShapeDtypeStruct((N,), jnp.uint32),
    scratch_types=[
        pltpu.VMEM((C,), jnp.uint32), pltpu.VMEM((C,), jnp.uint32),
        pltpu.VMEM((C,), jnp.float32), pltpu.VMEM((C,), jnp.float32),
        pltpu.VMEM((C,), jnp.float32), pltpu.VMEM((C,), jnp.float32),
        pltpu.VMEM((C,), jnp.uint32), pltpu.VMEM((C,), jnp.uint32),
        pltpu.VMEM((C,), jnp.uint32), pltpu.VMEM((C,), jnp.uint32),
        pltpu.VMEM_SHARED((K,), jnp.float32),
        pltpu.VMEM_SHARED((K,), jnp.uint32),
        pltpu.SemaphoreType.DMA, pltpu.SemaphoreType.DMA,
        pltpu.SemaphoreType.DMA, pltpu.SemaphoreType.DMA,
        pltpu.SemaphoreType.DMA, pltpu.SemaphoreType.DMA,
    ],
)(_sc_body)


def kernel(prob, u, kk, alias):
    kk32 = lax.convert_element_type(kk, jnp.uint32)
    alias32 = lax.convert_element_type(alias, jnp.uint32)
    out32 = _draw(prob, alias32, kk32, u)
    return lax.convert_element_type(out32, jnp.int64)


# speedup vs baseline: 233.8470x; 1.0012x over previous
import functools

import jax
import jax.numpy as jnp
from jax import lax
from jax.experimental import pallas as pl
from jax.experimental.pallas import tpu as pltpu
from jax.experimental.pallas import tpu_sc as plsc

N = 4194304
K = 100000
NC = 2
NS = 16
NW = NC * NS
NPW = N // NW
C = 8192
NCHUNK = NPW // C
L = 16
U = 8
P = 2000
NP = K // P


def _sc_body(prob_hbm, alias_hbm, kk_hbm, u_hbm, out_hbm,
             idx0, idx1, u0, u1, p0, p1, a0, a1, o0, o1,
             prob_sp, alias_sp,
             si0, si1, sg0, sg1, so0, so1):
    idx = (idx0, idx1)
    uu = (u0, u1)
    pp = (p0, p1)
    aa = (a0, a1)
    oo = (o0, o1)
    si = (si0, si1)
    sg = (sg0, sg1)
    so = (so0, so1)

    sid = lax.axis_index("s")
    wid = sid * jnp.int32(NC) + lax.axis_index("c")
    wbase = wid * jnp.int32(NPW)

    for g in range(NP):
        @pl.when(sid == jnp.int32(g % NS))
        def _():
            pltpu.sync_copy(prob_hbm.at[pl.ds(g * P, P)],
                            uu[0].at[pl.ds(0, P)])
            pltpu.sync_copy(uu[0].at[pl.ds(0, P)],
                            prob_sp.at[pl.ds(g * P, P)])
            pltpu.sync_copy(alias_hbm.at[pl.ds(g * P, P)],
                            idx[0].at[pl.ds(0, P)])
            pltpu.sync_copy(idx[0].at[pl.ds(0, P)],
                            alias_sp.at[pl.ds(g * P, P)])

    plsc.subcore_barrier()

    def base_of(g):
        return wbase + jnp.int32(g * C)

    def issue_in(g):
        b = g % 2
        cp_k = pltpu.make_async_copy(
            kk_hbm.at[pl.ds(base_of(g), C)], idx[b], si[b])
        cp_u = pltpu.make_async_copy(
            u_hbm.at[pl.ds(base_of(g), C)], uu[b], si[b])
        cp_k.start()
        cp_u.start()
        return cp_k, cp_u

    def issue_gather(g):
        b = g % 2
        cp_p = pltpu.make_async_copy(prob_sp.at[idx[b]], pp[b], sg[b])
        cp_a = pltpu.make_async_copy(alias_sp.at[idx[b]], aa[b], sg[b])
        cp_p.start()
        cp_a.start()
        return cp_p, cp_a

    def issue_out(g):
        b = g % 2
        cp_o = pltpu.make_async_copy(
            oo[b], out_hbm.at[pl.ds(base_of(g), C)], so[b])
        cp_o.start()
        return cp_o

    def compute(g):
        b = g % 2

        @plsc.parallel_loop(jnp.int32(0), jnp.int32(C), jnp.int32(L), unroll=U)
        def vec_body(off):
            p = pp[b][pl.ds(off, L)]
            av = aa[b][pl.ds(off, L)]
            kkv = idx[b][pl.ds(off, L)]
            uv = uu[b][pl.ds(off, L)]
            oo[b][pl.ds(off, L)] = jnp.where(uv < p, kkv, av)

    cp_in = {}
    cp_g = {}
    cp_out = {}

    cp_in[0] = issue_in(0)
    cp_in[1] = issue_in(1)
    cp_in[0][0].wait()
    cp_in[0][1].wait()
    cp_g[0] = issue_gather(0)

    for g in range(NCHUNK):
        if g + 1 < NCHUNK:
            cp_in[g + 1][0].wait()
            cp_in[g + 1][1].wait()
            cp_g[g + 1] = issue_gather(g + 1)
        cp_g[g][0].wait()
        cp_g[g][1].wait()
        if g >= 2:
            cp_out[g - 2].wait()
        compute(g)
        cp_out[g] = issue_out(g)
        if g + 2 < NCHUNK:
            cp_in[g + 2] = issue_in(g + 2)

    cp_out[NCHUNK - 2].wait()
    cp_out[NCHUNK - 1].wait()


_mesh = plsc.VectorSubcoreMesh(core_axis_name="c", subcore_axis_name="s")

_draw = functools.partial(
    pl.kernel,
    mesh=_mesh,
    out_type=jax.ShapeDtypeStruct((N,), jnp.uint32),
    scratch_types=[
        pltpu.VMEM((C,), jnp.uint32), pltpu.VMEM((C,), jnp.uint32),
        pltpu.VMEM((C,), jnp.float32), pltpu.VMEM((C,), jnp.float32),
        pltpu.VMEM((C,), jnp.float32), pltpu.VMEM((C,), jnp.float32),
        pltpu.VMEM((C,), jnp.uint32), pltpu.VMEM((C,), jnp.uint32),
        pltpu.VMEM((C,), jnp.uint32), pltpu.VMEM((C,), jnp.uint32),
        pltpu.VMEM_SHARED((K,), jnp.float32),
        pltpu.VMEM_SHARED((K,), jnp.uint32),
        pltpu.SemaphoreType.DMA, pltpu.SemaphoreType.DMA,
        pltpu.SemaphoreType.DMA, pltpu.SemaphoreType.DMA,
        pltpu.SemaphoreType.DMA, pltpu.SemaphoreType.DMA,
    ],
)(_sc_body)


def kernel(prob, u, kk, alias):
    kk32 = lax.convert_element_type(kk, jnp.uint32)
    alias32 = lax.convert_element_type(alias, jnp.uint32)
    out32 = _draw(prob, alias32, kk32, u)
    return lax.convert_element_type(out32, jnp.int64)
